# baseline (speedup 1.0000x reference)
import jax
import jax.numpy as jnp
from jax.experimental import pallas as pl
from jax.experimental.pallas import tpu as pltpu

B, S, D = 1, 2048, 1024
E, K, F = 8, 2, 512
FS = 2048
EPS = 1e-6
RM = 0.22
T = B * S
EPG = 2
NEG = E // EPG
GF = EPG * F
NSH = 4
FSC = FS // NSH
NSTEPS = NEG + NSH


def _fused_kernel(x_ref, rmsw_ref, gw_ref, wg_ref, wu_ref, wd_ref,
                  sg_ref, su_ref, sd_ref, o_ref,
                  hb_ref, comb_ref):
    j = pl.program_id(0)

    @pl.when(j == 0)
    def _init():
        x = x_ref[...]
        var = jnp.mean(x * x, axis=-1, keepdims=True)
        h = x * jax.lax.rsqrt(var + EPS) * rmsw_ref[...]
        logits = jnp.dot(h, gw_ref[...], preferred_element_type=jnp.float32)
        lcols = jax.lax.broadcasted_iota(jnp.int32, (T, E), 1)
        v1 = jnp.max(logits, axis=1, keepdims=True)
        i1 = jnp.argmax(logits, axis=1).reshape(T, 1)
        masked = jnp.where(lcols == i1, -jnp.inf, logits)
        v2 = jnp.max(masked, axis=1, keepdims=True)
        i2 = jnp.argmax(masked, axis=1).reshape(T, 1)
        p1 = jax.nn.sigmoid(v1 - v2)
        comb_ref[...] = (jnp.where(lcols == i1, p1, 0.0)
                         + jnp.where(lcols == i2, 1.0 - p1, 0.0))
        hb_ref[...] = h.astype(jnp.bfloat16)
        o_ref[...] = x

    @pl.when(j < NEG)
    def _experts():
        hb = hb_ref[...]
        g = jnp.dot(hb, wg_ref[0], preferred_element_type=jnp.float32)
        u = jnp.dot(hb, wu_ref[0], preferred_element_type=jnp.float32)
        comb = comb_ref[...]
        ecols = jax.lax.broadcasted_iota(jnp.int32, (T, E), 1)
        gcols = jax.lax.broadcasted_iota(jnp.int32, (T, GF), 1) // F
        scale = jnp.zeros((T, GF), jnp.float32)
        for k in range(EPG):
            wk = jnp.sum(jnp.where(ecols == j * EPG + k, comb, 0.0),
                         axis=1, keepdims=True)
            scale = jnp.where(gcols == k, wk, scale)
        inter = (jax.nn.silu(g) * u * scale).astype(jnp.bfloat16)
        o_ref[...] += jnp.dot(inter, wd_ref[0],
                              preferred_element_type=jnp.float32)

    @pl.when(j >= NEG)
    def _shared():
        hb = hb_ref[...]
        g = jnp.dot(hb, sg_ref[...], preferred_element_type=jnp.float32)
        u = jnp.dot(hb, su_ref[...], preferred_element_type=jnp.float32)
        inter = (jax.nn.silu(g) * u).astype(jnp.bfloat16)
        o_ref[...] += jnp.dot(inter, sd_ref[...],
                              preferred_element_type=jnp.float32)


def kernel(hidden_states, rms_w, gate_w, w_gate, w_up, w_down,
           sh_gate, sh_up, sh_down):
    x = hidden_states.reshape(T, D)
    gwt = gate_w.T
    wg = (w_gate.reshape(NEG, EPG, F, D).transpose(0, 3, 1, 2)
          .reshape(NEG, D, GF).astype(jnp.bfloat16))
    wu = (w_up.reshape(NEG, EPG, F, D).transpose(0, 3, 1, 2)
          .reshape(NEG, D, GF).astype(jnp.bfloat16))
    wd = ((RM * w_down).transpose(0, 2, 1).reshape(NEG, GF, D)
          .astype(jnp.bfloat16))
    sg = sh_gate.T.astype(jnp.bfloat16)
    su = sh_up.T.astype(jnp.bfloat16)
    sd = (RM * sh_down).T.astype(jnp.bfloat16)

    out = pl.pallas_call(
        _fused_kernel,
        grid=(NSTEPS,),
        in_specs=[
            pl.BlockSpec((T, D), lambda j: (0, 0)),
            pl.BlockSpec((1, D), lambda j: (0, 0)),
            pl.BlockSpec((D, E), lambda j: (0, 0)),
            pl.BlockSpec((1, D, GF), lambda j: (jnp.minimum(j, NEG - 1), 0, 0)),
            pl.BlockSpec((1, D, GF), lambda j: (jnp.minimum(j, NEG - 1), 0, 0)),
            pl.BlockSpec((1, GF, D), lambda j: (jnp.minimum(j, NEG - 1), 0, 0)),
            pl.BlockSpec((D, FSC), lambda j: (0, jnp.clip(j - NEG, 0, NSH - 1))),
            pl.BlockSpec((D, FSC), lambda j: (0, jnp.clip(j - NEG, 0, NSH - 1))),
            pl.BlockSpec((FSC, D), lambda j: (jnp.clip(j - NEG, 0, NSH - 1), 0)),
        ],
        out_specs=pl.BlockSpec((T, D), lambda j: (0, 0)),
        out_shape=jax.ShapeDtypeStruct((T, D), jnp.float32),
        scratch_shapes=[
            pltpu.VMEM((T, D), jnp.bfloat16),
            pltpu.VMEM((T, E), jnp.float32),
        ],
        compiler_params=pltpu.CompilerParams(
            dimension_semantics=("arbitrary",),
        ),
    )(x, rms_w.reshape(1, D), gwt, wg, wu, wd, sg, su, sd)
    return out.reshape(B, S, D)


# speedup vs baseline: 1.0070x; 1.0070x over previous
import jax
import jax.numpy as jnp
from jax.experimental import pallas as pl
from jax.experimental.pallas import tpu as pltpu

B, S, D = 1, 2048, 1024
E, K, F = 8, 2, 512
FS = 2048
EPS = 1e-6
RM = 0.22
T = B * S
NT = 2
TB = T // NT
EPG = 2
NEG = E // EPG
GF = EPG * F
NSH = 2
FSC = FS // NSH
NSTEPS = NEG + NSH


def _fused_kernel(x_ref, rmsw_ref, gw_ref, wg_ref, wu_ref, wd_ref,
                  sg_ref, su_ref, sd_ref, o_ref,
                  acc_ref, hb_ref, comb_ref):
    s = pl.program_id(1)

    @pl.when(s == 0)
    def _init():
        x = x_ref[...]
        var = jnp.mean(x * x, axis=-1, keepdims=True)
        h = x * jax.lax.rsqrt(var + EPS) * rmsw_ref[...]
        logits = jnp.dot(h, gw_ref[...], preferred_element_type=jnp.float32)
        lcols = jax.lax.broadcasted_iota(jnp.int32, (TB, E), 1)
        v1 = jnp.max(logits, axis=1, keepdims=True)
        i1 = jnp.argmax(logits, axis=1).reshape(TB, 1)
        masked = jnp.where(lcols == i1, -jnp.inf, logits)
        v2 = jnp.max(masked, axis=1, keepdims=True)
        i2 = jnp.argmax(masked, axis=1).reshape(TB, 1)
        p1 = jax.nn.sigmoid(v1 - v2)
        comb_ref[...] = (jnp.where(lcols == i1, p1, 0.0)
                         + jnp.where(lcols == i2, 1.0 - p1, 0.0))
        hb_ref[...] = h.astype(jnp.bfloat16)
        acc_ref[...] = jnp.zeros_like(acc_ref)

    @pl.when(s < NEG)
    def _experts():
        hb = hb_ref[...]
        comb = comb_ref[...]
        ecols = jax.lax.broadcasted_iota(jnp.int32, (TB, E), 1)
        for k in range(EPG):
            g = jnp.dot(hb, wg_ref[0][:, k * F:(k + 1) * F],
                        preferred_element_type=jnp.float32)
            u = jnp.dot(hb, wu_ref[0][:, k * F:(k + 1) * F],
                        preferred_element_type=jnp.float32)
            wk = jnp.sum(jnp.where(ecols == s * EPG + k, comb, 0.0),
                         axis=1, keepdims=True)
            inter = (jax.nn.silu(g) * u * wk).astype(jnp.bfloat16)
            acc_ref[...] += jnp.dot(inter, wd_ref[0][k * F:(k + 1) * F, :],
                                    preferred_element_type=jnp.float32)

    @pl.when(s >= NEG)
    def _shared():
        hb = hb_ref[...]
        for k in range(FSC // F):
            g = jnp.dot(hb, sg_ref[:, k * F:(k + 1) * F],
                        preferred_element_type=jnp.float32)
            u = jnp.dot(hb, su_ref[:, k * F:(k + 1) * F],
                        preferred_element_type=jnp.float32)
            inter = (jax.nn.silu(g) * u).astype(jnp.bfloat16)
            acc_ref[...] += jnp.dot(inter, sd_ref[k * F:(k + 1) * F, :],
                                    preferred_element_type=jnp.float32)

    @pl.when(s == NSTEPS - 1)
    def _fin():
        o_ref[...] = x_ref[...] + acc_ref[...]


def kernel(hidden_states, rms_w, gate_w, w_gate, w_up, w_down,
           sh_gate, sh_up, sh_down):
    x = hidden_states.reshape(T, D)
    gwt = gate_w.T
    wg = (w_gate.reshape(NEG, EPG, F, D).transpose(0, 3, 1, 2)
          .reshape(NEG, D, GF).astype(jnp.bfloat16))
    wu = (w_up.reshape(NEG, EPG, F, D).transpose(0, 3, 1, 2)
          .reshape(NEG, D, GF).astype(jnp.bfloat16))
    wd = ((RM * w_down).transpose(0, 2, 1).reshape(NEG, GF, D)
          .astype(jnp.bfloat16))
    sg = sh_gate.T.astype(jnp.bfloat16)
    su = sh_up.T.astype(jnp.bfloat16)
    sd = (RM * sh_down).T.astype(jnp.bfloat16)

    out = pl.pallas_call(
        _fused_kernel,
        grid=(NT, NSTEPS),
        in_specs=[
            pl.BlockSpec((TB, D), lambda t, s: (t, 0)),
            pl.BlockSpec((1, D), lambda t, s: (0, 0)),
            pl.BlockSpec((D, E), lambda t, s: (0, 0)),
            pl.BlockSpec((1, D, GF),
                         lambda t, s: (jnp.minimum(s, NEG - 1), 0, 0)),
            pl.BlockSpec((1, D, GF),
                         lambda t, s: (jnp.minimum(s, NEG - 1), 0, 0)),
            pl.BlockSpec((1, GF, D),
                         lambda t, s: (jnp.minimum(s, NEG - 1), 0, 0)),
            pl.BlockSpec((D, FSC),
                         lambda t, s: (0, jnp.clip(s - NEG, 0, NSH - 1))),
            pl.BlockSpec((D, FSC),
                         lambda t, s: (0, jnp.clip(s - NEG, 0, NSH - 1))),
            pl.BlockSpec((FSC, D),
                         lambda t, s: (jnp.clip(s - NEG, 0, NSH - 1), 0)),
        ],
        out_specs=pl.BlockSpec((TB, D), lambda t, s: (t, 0)),
        out_shape=jax.ShapeDtypeStruct((T, D), jnp.float32),
        scratch_shapes=[
            pltpu.VMEM((TB, D), jnp.float32),
            pltpu.VMEM((TB, D), jnp.bfloat16),
            pltpu.VMEM((TB, E), jnp.float32),
        ],
        compiler_params=pltpu.CompilerParams(
            dimension_semantics=("arbitrary", "arbitrary"),
        ),
    )(x, rms_w.reshape(1, D), gwt, wg, wu, wd, sg, su, sd)
    return out.reshape(B, S, D)


# speedup vs baseline: 1.3429x; 1.3336x over previous
import jax
import jax.numpy as jnp
from jax.experimental import pallas as pl
from jax.experimental.pallas import tpu as pltpu

B, S, D = 1, 2048, 1024
E, K, F = 8, 2, 512
FS = 2048
EPS = 1e-6
RM = 0.22
T = B * S
NSH = 4
FSC = FS // NSH
NSTEPS = E + NSH


def _fused_kernel(x_ref, rmsw_ref, gw_ref, wg_ref, wu_ref, wd_ref,
                  sg_ref, su_ref, sd_ref, o_ref,
                  acc_ref, hb_ref, comb_ref):
    j = pl.program_id(0)

    @pl.when(j == 0)
    def _init():
        x = x_ref[...]
        var = jnp.mean(x * x, axis=-1, keepdims=True)
        h = x * jax.lax.rsqrt(var + EPS) * rmsw_ref[...]
        logits = jnp.dot(h, gw_ref[...], preferred_element_type=jnp.float32)
        lcols = jax.lax.broadcasted_iota(jnp.int32, (T, E), 1)
        v1 = jnp.max(logits, axis=1, keepdims=True)
        i1 = jnp.argmax(logits, axis=1).reshape(T, 1)
        masked = jnp.where(lcols == i1, -jnp.inf, logits)
        v2 = jnp.max(masked, axis=1, keepdims=True)
        i2 = jnp.argmax(masked, axis=1).reshape(T, 1)
        p1 = jax.nn.sigmoid(v1 - v2)
        comb_ref[...] = (jnp.where(lcols == i1, p1, 0.0)
                         + jnp.where(lcols == i2, 1.0 - p1, 0.0))
        hb_ref[...] = h.astype(jnp.bfloat16)
        acc_ref[...] = jnp.zeros_like(acc_ref)

    @pl.when(j < E)
    def _expert():
        hb = hb_ref[...]
        g = jnp.dot(hb, wg_ref[0], preferred_element_type=jnp.float32)
        u = jnp.dot(hb, wu_ref[0], preferred_element_type=jnp.float32)
        cols = jax.lax.broadcasted_iota(jnp.int32, (T, E), 1)
        w = jnp.sum(jnp.where(cols == j, comb_ref[...], 0.0),
                    axis=1, keepdims=True)
        inter = (jax.nn.silu(g) * u * w).astype(jnp.bfloat16)
        acc_ref[...] += jnp.dot(inter, wd_ref[0],
                                preferred_element_type=jnp.float32)

    @pl.when(j >= E)
    def _shared():
        hb = hb_ref[...]
        g = jnp.dot(hb, sg_ref[...], preferred_element_type=jnp.float32)
        u = jnp.dot(hb, su_ref[...], preferred_element_type=jnp.float32)
        inter = (jax.nn.silu(g) * u).astype(jnp.bfloat16)
        acc_ref[...] += jnp.dot(inter, sd_ref[...],
                                preferred_element_type=jnp.float32)

    @pl.when(j == NSTEPS - 1)
    def _fin():
        o_ref[...] = x_ref[...] + RM * acc_ref[...]


def kernel(hidden_states, rms_w, gate_w, w_gate, w_up, w_down,
           sh_gate, sh_up, sh_down):
    x = hidden_states.reshape(T, D)
    gwt = gate_w.T
    wg = w_gate.transpose(0, 2, 1).astype(jnp.bfloat16)
    wu = w_up.transpose(0, 2, 1).astype(jnp.bfloat16)
    wd = w_down.transpose(0, 2, 1).astype(jnp.bfloat16)
    sg = sh_gate.T.astype(jnp.bfloat16)
    su = sh_up.T.astype(jnp.bfloat16)
    sd = sh_down.T.astype(jnp.bfloat16)

    out = pl.pallas_call(
        _fused_kernel,
        grid=(NSTEPS,),
        in_specs=[
            pl.BlockSpec((T, D), lambda j: (0, 0)),
            pl.BlockSpec((1, D), lambda j: (0, 0)),
            pl.BlockSpec((D, E), lambda j: (0, 0)),
            pl.BlockSpec((1, D, F), lambda j: (jnp.minimum(j, E - 1), 0, 0)),
            pl.BlockSpec((1, D, F), lambda j: (jnp.minimum(j, E - 1), 0, 0)),
            pl.BlockSpec((1, F, D), lambda j: (jnp.minimum(j, E - 1), 0, 0)),
            pl.BlockSpec((D, FSC), lambda j: (0, jnp.clip(j - E, 0, NSH - 1))),
            pl.BlockSpec((D, FSC), lambda j: (0, jnp.clip(j - E, 0, NSH - 1))),
            pl.BlockSpec((FSC, D), lambda j: (jnp.clip(j - E, 0, NSH - 1), 0)),
        ],
        out_specs=pl.BlockSpec((T, D), lambda j: (0, 0)),
        out_shape=jax.ShapeDtypeStruct((T, D), jnp.float32),
        scratch_shapes=[
            pltpu.VMEM((T, D), jnp.float32),
            pltpu.VMEM((T, D), jnp.bfloat16),
            pltpu.VMEM((T, E), jnp.float32),
        ],
        compiler_params=pltpu.CompilerParams(
            dimension_semantics=("arbitrary",),
        ),
    )(x, rms_w.reshape(1, D), gwt, wg, wu, wd, sg, su, sd)
    return out.reshape(B, S, D)


# speedup vs baseline: 1.6372x; 1.2191x over previous
import jax
import jax.numpy as jnp
from jax.experimental import pallas as pl
from jax.experimental.pallas import tpu as pltpu

B, S, D = 1, 2048, 1024
E, K, F = 8, 2, 512
FS = 2048
EPS = 1e-6
RM = 0.22
T = B * S
NSH = 4
FSC = FS // NSH
NSTEPS = E + NSH

_TDOT = (((1,), (1,)), ((), ()))


def _fused_kernel(x_ref, rmsw_ref, gw_ref, wg_ref, wu_ref, wd_ref,
                  sg_ref, su_ref, sd_ref, o_ref,
                  acc_ref, hb_ref, comb_ref):
    j = pl.program_id(0)

    @pl.when(j == 0)
    def _init():
        x = x_ref[...]
        var = jnp.mean(x * x, axis=-1, keepdims=True)
        h = x * jax.lax.rsqrt(var + EPS) * rmsw_ref[...]
        logits = jax.lax.dot_general(h, gw_ref[...], _TDOT,
                                     preferred_element_type=jnp.float32)
        lcols = jax.lax.broadcasted_iota(jnp.int32, (T, E), 1)
        v1 = jnp.max(logits, axis=1, keepdims=True)
        i1 = jnp.argmax(logits, axis=1).reshape(T, 1)
        masked = jnp.where(lcols == i1, -jnp.inf, logits)
        v2 = jnp.max(masked, axis=1, keepdims=True)
        i2 = jnp.argmax(masked, axis=1).reshape(T, 1)
        p1 = jax.nn.sigmoid(v1 - v2)
        comb_ref[...] = (jnp.where(lcols == i1, p1, 0.0)
                         + jnp.where(lcols == i2, 1.0 - p1, 0.0))
        hb_ref[...] = h.astype(jnp.bfloat16)
        acc_ref[...] = jnp.zeros_like(acc_ref)

    @pl.when(j < E)
    def _expert():
        hb = hb_ref[...]
        g = jax.lax.dot_general(hb, wg_ref[0], _TDOT,
                                preferred_element_type=jnp.float32)
        u = jax.lax.dot_general(hb, wu_ref[0], _TDOT,
                                preferred_element_type=jnp.float32)
        cols = jax.lax.broadcasted_iota(jnp.int32, (T, E), 1)
        w = jnp.sum(jnp.where(cols == j, comb_ref[...], 0.0),
                    axis=1, keepdims=True)
        inter = (jax.nn.silu(g) * u * w).astype(jnp.bfloat16)
        acc_ref[...] += jax.lax.dot_general(
            inter, wd_ref[0], _TDOT, preferred_element_type=jnp.float32)

    @pl.when(j >= E)
    def _shared():
        hb = hb_ref[...]
        g = jax.lax.dot_general(hb, sg_ref[...], _TDOT,
                                preferred_element_type=jnp.float32)
        u = jax.lax.dot_general(hb, su_ref[...], _TDOT,
                                preferred_element_type=jnp.float32)
        inter = (jax.nn.silu(g) * u).astype(jnp.bfloat16)
        acc_ref[...] += jax.lax.dot_general(
            inter, sd_ref[...], _TDOT, preferred_element_type=jnp.float32)

    @pl.when(j == NSTEPS - 1)
    def _fin():
        o_ref[...] = x_ref[...] + RM * acc_ref[...]


def kernel(hidden_states, rms_w, gate_w, w_gate, w_up, w_down,
           sh_gate, sh_up, sh_down):
    x = hidden_states.reshape(T, D)
    wg = w_gate.astype(jnp.bfloat16)
    wu = w_up.astype(jnp.bfloat16)
    wd = w_down.astype(jnp.bfloat16)
    sg = sh_gate.astype(jnp.bfloat16)
    su = sh_up.astype(jnp.bfloat16)
    sd = sh_down.astype(jnp.bfloat16)

    out = pl.pallas_call(
        _fused_kernel,
        grid=(NSTEPS,),
        in_specs=[
            pl.BlockSpec((T, D), lambda j: (0, 0)),
            pl.BlockSpec((1, D), lambda j: (0, 0)),
            pl.BlockSpec((E, D), lambda j: (0, 0)),
            pl.BlockSpec((1, F, D), lambda j: (jnp.minimum(j, E - 1), 0, 0)),
            pl.BlockSpec((1, F, D), lambda j: (jnp.minimum(j, E - 1), 0, 0)),
            pl.BlockSpec((1, D, F), lambda j: (jnp.minimum(j, E - 1), 0, 0)),
            pl.BlockSpec((FSC, D), lambda j: (jnp.clip(j - E, 0, NSH - 1), 0)),
            pl.BlockSpec((FSC, D), lambda j: (jnp.clip(j - E, 0, NSH - 1), 0)),
            pl.BlockSpec((D, FSC), lambda j: (0, jnp.clip(j - E, 0, NSH - 1))),
        ],
        out_specs=pl.BlockSpec((T, D), lambda j: (0, 0)),
        out_shape=jax.ShapeDtypeStruct((T, D), jnp.float32),
        scratch_shapes=[
            pltpu.VMEM((T, D), jnp.float32),
            pltpu.VMEM((T, D), jnp.bfloat16),
            pltpu.VMEM((T, E), jnp.float32),
        ],
        compiler_params=pltpu.CompilerParams(
            dimension_semantics=("arbitrary",),
        ),
    )(x, rms_w.reshape(1, D), gate_w, wg, wu, wd, sg, su, sd)
    return out.reshape(B, S, D)
